# baseline (speedup 1.0000x reference)
import jax
import jax.numpy as jnp
import numpy as np
from jax.experimental import pallas as pl
from jax.experimental.pallas import tpu as pltpu

_LOG2E = 1.4426950408889634
_SPLIT = 11


def _rounds(x0, x1, rots):
    for r in rots:
        x0 = x0 + x1
        x1 = (x1 << np.uint32(r)) | (x1 >> np.uint32(32 - r))
        x1 = x1 ^ x0
    return x0, x1


def _threefry_u(flat):
    ks0 = jnp.uint32(0)
    ks1 = jnp.uint32(42)
    ks2 = jnp.uint32(0 ^ 42 ^ 0x1BD11BDA)
    r1 = (13, 15, 26, 6)
    r2 = (17, 29, 16, 24)
    x0 = jnp.zeros_like(flat) + ks0
    x1 = flat + ks1
    x0, x1 = _rounds(x0, x1, r1)
    x0 = x0 + ks1
    x1 = x1 + ks2 + np.uint32(1)
    x0, x1 = _rounds(x0, x1, r2)
    x0 = x0 + ks2
    x1 = x1 + ks0 + np.uint32(2)
    x0, x1 = _rounds(x0, x1, r1)
    x0 = x0 + ks0
    x1 = x1 + ks1 + np.uint32(3)
    x0, x1 = _rounds(x0, x1, r2)
    x0 = x0 + ks1
    x1 = x1 + ks2 + np.uint32(4)
    x0, x1 = _rounds(x0, x1, r1)
    x0 = x0 + ks2
    x1 = x1 + ks0 + np.uint32(5)
    bits = x0 ^ x1
    eps = np.float32(np.finfo(np.float32).eps)
    span = (np.float32(1.0) - eps) - eps
    f = jax.lax.bitcast_convert_type(
        (bits >> np.uint32(9)) | jnp.uint32(0x3F800000), jnp.float32) - 1.0
    return jnp.maximum(eps, f * span + eps)


def _body(params_ref, x_ref, eg_ref, emb_ref,
          q_ref, loss_ref, perp_ref,
          hist_ref, sse_ref, ent_ref, ge_ref):
    i = pl.program_id(0)
    nb = pl.num_programs(0)
    c = 0.5 * params_ref[0, 0]
    inv_temp = params_ref[0, 1]
    fast = inv_temp == 1.0

    xb = x_ref[0]
    emb = emb_ref[...]
    M = emb.shape[0]
    T = xb.shape[1]
    emb2c = (c + c) * emb
    ce2 = 0.5 * jnp.sum(emb2c * emb, axis=1, keepdims=True)
    xy2 = jnp.dot(emb2c, xb, preferred_element_type=jnp.float32,
                  precision=jax.lax.Precision.HIGHEST)
    ncr = xy2 - ce2

    nmax = jnp.max(ncr, axis=0, keepdims=True)
    t = ncr - nmax
    ex = jnp.exp2(t * _LOG2E)
    s = jnp.sum(ex, axis=0, keepdims=True)
    sxt = jnp.sum(ex * t, axis=0, keepdims=True)
    entp = sxt * (1.0 / s) - jnp.log(s)

    streamed = i < _SPLIT

    @pl.when(jnp.logical_and(fast, streamed))
    def _ge_fast_streamed():
        ge_ref[...] = ex * eg_ref[0]

    @pl.when(jnp.logical_and(fast, jnp.logical_not(streamed)))
    def _ge_fast_generated():
        m_iota = jax.lax.broadcasted_iota(jnp.int32, (M, T), 0)
        t_iota = jax.lax.broadcasted_iota(jnp.int32, (M, T), 1)
        flat = (i * (M * T) + t_iota * M + m_iota).astype(jnp.uint32)
        u = _threefry_u(flat)
        ge_ref[...] = ex * (-1.0 / jnp.log(u))

    @pl.when(jnp.logical_not(fast))
    def _ge_general():
        @pl.when(streamed)
        def _g_streamed():
            ge_ref[...] = jnp.log(eg_ref[0])

        @pl.when(jnp.logical_not(streamed))
        def _g_generated():
            m_iota = jax.lax.broadcasted_iota(jnp.int32, (M, T), 0)
            t_iota = jax.lax.broadcasted_iota(jnp.int32, (M, T), 1)
            flat = (i * (M * T) + t_iota * M + m_iota).astype(jnp.uint32)
            u = _threefry_u(flat)
            ge_ref[...] = -jnp.log(-jnp.log(u))

        gl = (ncr + ge_ref[...]) * inv_temp
        gmx = jnp.max(gl, axis=0, keepdims=True)
        ge_ref[...] = jnp.exp2((gl - gmx) * _LOG2E)

    ge = ge_ref[...]
    gs = jnp.sum(ge, axis=0, keepdims=True)
    q = jax.lax.dot_general(emb, ge, (((0,), (0,)), ((), ())),
                            preferred_element_type=jnp.float32)
    q = q * (1.0 / gs)
    q_ref[0] = q

    hpart = jnp.sum((ncr == nmax).astype(jnp.float32), axis=1,
                    keepdims=True)
    ssep = jnp.sum((xb - q) ** 2, axis=0, keepdims=True)

    @pl.when(i == 0)
    def _init():
        hist_ref[...] = jnp.zeros_like(hist_ref)
        ent_ref[...] = jnp.zeros_like(ent_ref)
        sse_ref[...] = jnp.zeros_like(sse_ref)

    hist_ref[...] += hpart
    ent_ref[...] += entp
    sse_ref[...] += ssep

    @pl.when(i == nb - 1)
    def _finish():
        n_tok = nb * T
        avg = hist_ref[...] / jnp.float32(n_tok)
        perp = jnp.exp(-jnp.sum(avg * jnp.log(avg + 1e-10)))
        sse = jnp.sum(sse_ref[...])
        ent = jnp.sum(ent_ref[...])
        loss_ref[0, 0] = (c * sse + ent) / jnp.float32(nb)
        perp_ref[0, 0] = perp


def kernel(x, temperature, embedding, log_var_q):
    B, D, T = x.shape
    M, _ = embedding.shape
    eps = jnp.finfo(jnp.float32).eps
    u = jax.random.uniform(jax.random.key(42), (_SPLIT * T, M), jnp.float32,
                           minval=eps, maxval=1.0 - eps)
    eg = jnp.transpose((-1.0 / jnp.log(u)).reshape(_SPLIT, T, M), (0, 2, 1))

    precision = jnp.exp(-log_var_q).astype(jnp.float32)
    inv_temp = (1.0 / temperature).astype(jnp.float32)
    params = jnp.stack([precision, inv_temp]).reshape(1, 2)

    q, loss, perp = pl.pallas_call(
        _body,
        grid=(B,),
        in_specs=[
            pl.BlockSpec((1, 2), lambda i: (0, 0), memory_space=pltpu.SMEM),
            pl.BlockSpec((1, D, T), lambda i: (i, 0, 0)),
            pl.BlockSpec((1, M, T),
                         lambda i: (jnp.minimum(i, _SPLIT - 1), 0, 0)),
            pl.BlockSpec((M, D), lambda i: (0, 0)),
        ],
        out_specs=[
            pl.BlockSpec((1, D, T), lambda i: (i, 0, 0)),
            pl.BlockSpec((1, 1), lambda i: (0, 0), memory_space=pltpu.SMEM),
            pl.BlockSpec((1, 1), lambda i: (0, 0), memory_space=pltpu.SMEM),
        ],
        out_shape=[
            jax.ShapeDtypeStruct((B, D, T), jnp.float32),
            jax.ShapeDtypeStruct((1, 1), jnp.float32),
            jax.ShapeDtypeStruct((1, 1), jnp.float32),
        ],
        scratch_shapes=[
            pltpu.VMEM((M, 1), jnp.float32),
            pltpu.VMEM((1, T), jnp.float32),
            pltpu.VMEM((1, T), jnp.float32),
            pltpu.VMEM((M, T), jnp.float32),
        ],
        compiler_params=pltpu.CompilerParams(
            dimension_semantics=("arbitrary",)),
    )(params, x, eg, embedding)

    return q, loss[0, 0], perp[0, 0]


# speedup vs baseline: 1.0112x; 1.0112x over previous
import jax
import jax.numpy as jnp
from jax.experimental import pallas as pl
from jax.experimental.pallas import tpu as pltpu

_LOG2E = 1.4426950408889634


def _body(params_ref, x_ref, eg_ref, emb_ref,
          q_ref, loss_ref, perp_ref,
          hist_ref, sse_ref, ent_ref, ge_ref):
    i = pl.program_id(0)
    nb = pl.num_programs(0)
    c = 0.5 * params_ref[0, 0]
    inv_temp = params_ref[0, 1]
    fast = inv_temp == 1.0

    xb = x_ref[0]
    emb = emb_ref[...]
    T = xb.shape[1]
    emb2c = (c + c) * emb
    ce2 = 0.5 * jnp.sum(emb2c * emb, axis=1, keepdims=True)
    xy2 = jnp.dot(emb2c, xb, preferred_element_type=jnp.float32,
                  precision=jax.lax.Precision.HIGHEST)
    ncr = xy2 - ce2

    nmax = jnp.max(ncr, axis=0, keepdims=True)
    t = ncr - nmax
    ex = jnp.exp2(t * _LOG2E)
    s = jnp.sum(ex, axis=0, keepdims=True)
    sxt = jnp.sum(ex * t, axis=0, keepdims=True)
    entp = sxt * (1.0 / s) - jnp.log(s)

    @pl.when(fast)
    def _ge_fast():
        ge_ref[...] = ex * eg_ref[0].astype(jnp.float32)

    @pl.when(jnp.logical_not(fast))
    def _ge_general():
        gl = (ncr + jnp.log(eg_ref[0].astype(jnp.float32))) * inv_temp
        gmx = jnp.max(gl, axis=0, keepdims=True)
        ge_ref[...] = jnp.exp2((gl - gmx) * _LOG2E)

    ge = ge_ref[...]
    gs = jnp.sum(ge, axis=0, keepdims=True)
    q = jax.lax.dot_general(emb, ge, (((0,), (0,)), ((), ())),
                            preferred_element_type=jnp.float32)
    q = q * (1.0 / gs)
    q_ref[0] = q

    hpart = jnp.sum((ncr == nmax).astype(jnp.float32), axis=1,
                    keepdims=True)
    ssep = jnp.sum((xb - q) ** 2, axis=0, keepdims=True)

    @pl.when(i == 0)
    def _init():
        hist_ref[...] = jnp.zeros_like(hist_ref)
        ent_ref[...] = jnp.zeros_like(ent_ref)
        sse_ref[...] = jnp.zeros_like(sse_ref)

    hist_ref[...] += hpart
    ent_ref[...] += entp
    sse_ref[...] += ssep

    @pl.when(i == nb - 1)
    def _finish():
        n_tok = nb * T
        avg = hist_ref[...] / jnp.float32(n_tok)
        perp = jnp.exp(-jnp.sum(avg * jnp.log(avg + 1e-10)))
        sse = jnp.sum(sse_ref[...])
        ent = jnp.sum(ent_ref[...])
        loss_ref[0, 0] = (c * sse + ent) / jnp.float32(nb)
        perp_ref[0, 0] = perp


def kernel(x, temperature, embedding, log_var_q):
    B, D, T = x.shape
    M, _ = embedding.shape
    eps = jnp.finfo(jnp.float32).eps
    u = jax.random.uniform(jax.random.key(42), (B * T, M), jnp.float32,
                           minval=eps, maxval=1.0 - eps)
    eg = jnp.transpose((-1.0 / jnp.log(u)).reshape(B, T, M),
                       (0, 2, 1)).astype(jnp.bfloat16)

    precision = jnp.exp(-log_var_q).astype(jnp.float32)
    inv_temp = (1.0 / temperature).astype(jnp.float32)
    params = jnp.stack([precision, inv_temp]).reshape(1, 2)

    q, loss, perp = pl.pallas_call(
        _body,
        grid=(B,),
        in_specs=[
            pl.BlockSpec((1, 2), lambda i: (0, 0), memory_space=pltpu.SMEM),
            pl.BlockSpec((1, D, T), lambda i: (i, 0, 0)),
            pl.BlockSpec((1, M, T), lambda i: (i, 0, 0)),
            pl.BlockSpec((M, D), lambda i: (0, 0)),
        ],
        out_specs=[
            pl.BlockSpec((1, D, T), lambda i: (i, 0, 0)),
            pl.BlockSpec((1, 1), lambda i: (0, 0), memory_space=pltpu.SMEM),
            pl.BlockSpec((1, 1), lambda i: (0, 0), memory_space=pltpu.SMEM),
        ],
        out_shape=[
            jax.ShapeDtypeStruct((B, D, T), jnp.float32),
            jax.ShapeDtypeStruct((1, 1), jnp.float32),
            jax.ShapeDtypeStruct((1, 1), jnp.float32),
        ],
        scratch_shapes=[
            pltpu.VMEM((M, 1), jnp.float32),
            pltpu.VMEM((1, T), jnp.float32),
            pltpu.VMEM((1, T), jnp.float32),
            pltpu.VMEM((M, T), jnp.float32),
        ],
        compiler_params=pltpu.CompilerParams(
            dimension_semantics=("arbitrary",)),
    )(params, x, eg, embedding)

    return q, loss[0, 0], perp[0, 0]


# speedup vs baseline: 1.1109x; 1.0986x over previous
import jax
import jax.numpy as jnp
from jax.experimental import pallas as pl
from jax.experimental.pallas import tpu as pltpu

_LOG2E = 1.4426950408889634
_G = 4


def _body(params_ref, x_ref, eg_ref, emb_ref,
          q_ref, loss_ref, perp_ref,
          hist_ref, sse_ref, ent_ref, ge_ref):
    i = pl.program_id(0)
    nb = pl.num_programs(0)
    c = 0.5 * params_ref[0, 0]
    inv_temp = params_ref[0, 1]
    fast = inv_temp == 1.0

    xb = jnp.concatenate([x_ref[g] for g in range(_G)], axis=1)
    T = xb.shape[1] // _G
    emb = emb_ref[...]
    emb2c = (c + c) * emb
    ce2 = 0.5 * jnp.sum(emb2c * emb, axis=1, keepdims=True)
    xy2 = jnp.dot(emb2c, xb, preferred_element_type=jnp.float32,
                  precision=jax.lax.Precision.HIGHEST)
    ncr = xy2 - ce2

    nmax = jnp.max(ncr, axis=0, keepdims=True)
    t = ncr - nmax
    ex = jnp.exp2(t * _LOG2E)
    s = jnp.sum(ex, axis=0, keepdims=True)
    sxt = jnp.sum(ex * t, axis=0, keepdims=True)
    entp = sxt * (1.0 / s) - jnp.log(s)

    @pl.when(fast)
    def _ge_fast():
        ge_ref[...] = ex * eg_ref[0].astype(jnp.float32)

    @pl.when(jnp.logical_not(fast))
    def _ge_general():
        gl = (ncr + jnp.log(eg_ref[0].astype(jnp.float32))) * inv_temp
        gmx = jnp.max(gl, axis=0, keepdims=True)
        ge_ref[...] = jnp.exp2((gl - gmx) * _LOG2E)

    ge = ge_ref[...]
    gs = jnp.sum(ge, axis=0, keepdims=True)
    q = jax.lax.dot_general(emb, ge, (((0,), (0,)), ((), ())),
                            preferred_element_type=jnp.float32)
    q = q * (1.0 / gs)
    for g in range(_G):
        q_ref[g] = q[:, g * T:(g + 1) * T]

    hpart = jnp.sum((ncr == nmax).astype(jnp.float32), axis=1,
                    keepdims=True)
    ssep = jnp.sum((xb - q) ** 2, axis=0, keepdims=True)

    @pl.when(i == 0)
    def _init():
        hist_ref[...] = jnp.zeros_like(hist_ref)
        ent_ref[...] = jnp.zeros_like(ent_ref)
        sse_ref[...] = jnp.zeros_like(sse_ref)

    hist_ref[...] += hpart
    ent_ref[...] += entp
    sse_ref[...] += ssep

    @pl.when(i == nb - 1)
    def _finish():
        n_tok = nb * _G * T
        avg = hist_ref[...] / jnp.float32(n_tok)
        perp = jnp.exp(-jnp.sum(avg * jnp.log(avg + 1e-10)))
        sse = jnp.sum(sse_ref[...])
        ent = jnp.sum(ent_ref[...])
        loss_ref[0, 0] = (c * sse + ent) / jnp.float32(nb * _G)
        perp_ref[0, 0] = perp


def kernel(x, temperature, embedding, log_var_q):
    B, D, T = x.shape
    M, _ = embedding.shape
    S = B // _G
    eps = jnp.finfo(jnp.float32).eps
    u = jax.random.uniform(jax.random.key(42), (B * T, M), jnp.float32,
                           minval=eps, maxval=1.0 - eps)
    eg = (-1.0 / jnp.log(u)).reshape(S, _G, T, M)
    eg = jnp.transpose(eg, (0, 3, 1, 2)).reshape(S, M, _G * T)
    eg = eg.astype(jnp.bfloat16)

    precision = jnp.exp(-log_var_q).astype(jnp.float32)
    inv_temp = (1.0 / temperature).astype(jnp.float32)
    params = jnp.stack([precision, inv_temp]).reshape(1, 2)

    q, loss, perp = pl.pallas_call(
        _body,
        grid=(S,),
        in_specs=[
            pl.BlockSpec((1, 2), lambda i: (0, 0), memory_space=pltpu.SMEM),
            pl.BlockSpec((_G, D, T), lambda i: (i, 0, 0)),
            pl.BlockSpec((1, M, _G * T), lambda i: (i, 0, 0)),
            pl.BlockSpec((M, D), lambda i: (0, 0)),
        ],
        out_specs=[
            pl.BlockSpec((_G, D, T), lambda i: (i, 0, 0)),
            pl.BlockSpec((1, 1), lambda i: (0, 0), memory_space=pltpu.SMEM),
            pl.BlockSpec((1, 1), lambda i: (0, 0), memory_space=pltpu.SMEM),
        ],
        out_shape=[
            jax.ShapeDtypeStruct((B, D, T), jnp.float32),
            jax.ShapeDtypeStruct((1, 1), jnp.float32),
            jax.ShapeDtypeStruct((1, 1), jnp.float32),
        ],
        scratch_shapes=[
            pltpu.VMEM((M, 1), jnp.float32),
            pltpu.VMEM((1, _G * T), jnp.float32),
            pltpu.VMEM((1, _G * T), jnp.float32),
            pltpu.VMEM((M, _G * T), jnp.float32),
        ],
        compiler_params=pltpu.CompilerParams(
            dimension_semantics=("arbitrary",)),
    )(params, x, eg, embedding)

    return q, loss[0, 0], perp[0, 0]
